# Initial kernel scaffold; baseline (speedup 1.0000x reference)
import jax
import jax.numpy as jnp
from jax.experimental import pallas as pl


def kernel(xyz1, xyz2, points1, points2, W1, b1, g1, be1, W2, b2, g2, be2):
    raise NotImplementedError("write your pallas kernel here")


# speedup vs baseline: 19.3530x; 19.3530x over previous
import jax
import jax.numpy as jnp
from jax.experimental import pallas as pl


def _k1_body(x1_ref, x2t_ref, x1sq_ref, x2sq_ref, p1_ref, p2_ref,
             w1_ref, b1_ref, h1_ref, sum1_ref, sq1_ref):
    b = pl.program_id(0)
    nt = pl.program_id(1)
    x1 = x1_ref[0]
    x2t = x2t_ref[0]
    mm = jax.lax.dot_general(x1, x2t, (((1,), (0,)), ((), ())),
                             preferred_element_type=jnp.float32)
    d = -2.0 * mm
    d = d + x1sq_ref[0][:, 0:1]
    d = d + x2sq_ref[0][0:1, :]
    s_dim = d.shape[1]
    iota = jax.lax.broadcasted_iota(jnp.int32, d.shape, 1)
    dvals = []
    idxs = []
    dd = d
    for _ in range(3):
        m = jnp.min(dd, axis=1, keepdims=True)
        isel = jnp.min(jnp.where(dd == m, iota, s_dim), axis=1,
                       keepdims=True)
        dvals.append(m)
        idxs.append(isel)
        dd = jnp.where(iota == isel, jnp.float32(jnp.inf), dd)
    r0 = 1.0 / (dvals[0] + 1e-8)
    r1 = 1.0 / (dvals[1] + 1e-8)
    r2 = 1.0 / (dvals[2] + 1e-8)
    norm = r0 + r1 + r2
    oh = (jnp.where(iota == idxs[0], r0 / norm, 0.0)
          + jnp.where(iota == idxs[1], r1 / norm, 0.0)
          + jnp.where(iota == idxs[2], r2 / norm, 0.0))
    p2 = p2_ref[0]
    interp_t = jax.lax.dot_general(p2, oh, (((1,), (1,)), ((), ())),
                                   preferred_element_type=jnp.float32)
    x = jnp.concatenate([p1_ref[0], interp_t], axis=0)
    h = jax.lax.dot_general(w1_ref[...], x, (((1,), (0,)), ((), ())),
                            preferred_element_type=jnp.float32)
    h = h + b1_ref[:, 0:1]
    h1_ref[0] = h
    first = jnp.logical_and(b == 0, nt == 0)

    @pl.when(first)
    def _():
        sum1_ref[...] = jnp.zeros_like(sum1_ref)
        sq1_ref[...] = jnp.zeros_like(sq1_ref)

    sum1_ref[...] += jnp.sum(h, axis=1, keepdims=True)
    sq1_ref[...] += jnp.sum(h * h, axis=1, keepdims=True)


def _k2_body(h1_ref, s1_ref, t1_ref, w2_ref, b2_ref,
             h2_ref, sum2_ref, sq2_ref):
    b = pl.program_id(0)
    nt = pl.program_id(1)
    h1 = h1_ref[0]
    a = jnp.maximum(h1 * s1_ref[:, 0:1] + t1_ref[:, 0:1], 0.0)
    h2 = jax.lax.dot_general(w2_ref[...], a, (((1,), (0,)), ((), ())),
                             preferred_element_type=jnp.float32)
    h2 = h2 + b2_ref[:, 0:1]
    h2_ref[0] = h2
    first = jnp.logical_and(b == 0, nt == 0)

    @pl.when(first)
    def _():
        sum2_ref[...] = jnp.zeros_like(sum2_ref)
        sq2_ref[...] = jnp.zeros_like(sq2_ref)

    sum2_ref[...] += jnp.sum(h2, axis=1, keepdims=True)
    sq2_ref[...] += jnp.sum(h2 * h2, axis=1, keepdims=True)


def _k3_body(h2_ref, s2_ref, t2_ref, o_ref):
    o_ref[0] = jnp.maximum(h2_ref[0] * s2_ref[:, 0:1] + t2_ref[:, 0:1], 0.0)


def _col(v):
    return jnp.broadcast_to(v[:, None], (v.shape[0], 128)).astype(jnp.float32)


def kernel(xyz1, xyz2, points1, points2, W1, b1, g1, be1, W2, b2, g2, be2):
    B, N, _ = xyz1.shape
    S = xyz2.shape[1]
    C1 = points1.shape[1]
    C2 = points2.shape[1]
    H1 = W1.shape[0]
    IN_CH = W1.shape[1]
    H2 = W2.shape[0]
    NT = 512 if N % 512 == 0 else N
    NB = N // NT
    eps = 1e-5
    cnt = B * N

    pad = [(0, 0), (0, 0), (0, 5)]
    x1p = jnp.pad(xyz1, pad)
    x2tp = jnp.pad(xyz2, pad).transpose(0, 2, 1)
    x1sq = jnp.broadcast_to(jnp.sum(xyz1 ** 2, axis=-1)[:, :, None],
                            (B, N, 8))
    x2sq = jnp.broadcast_to(jnp.sum(xyz2 ** 2, axis=-1)[:, None, :],
                            (B, 8, S))

    h1, s1sum, s1sq = pl.pallas_call(
        _k1_body,
        grid=(B, NB),
        in_specs=[
            pl.BlockSpec((1, NT, 8), lambda b, n: (b, n, 0)),
            pl.BlockSpec((1, 8, S), lambda b, n: (b, 0, 0)),
            pl.BlockSpec((1, NT, 8), lambda b, n: (b, n, 0)),
            pl.BlockSpec((1, 8, S), lambda b, n: (b, 0, 0)),
            pl.BlockSpec((1, C1, NT), lambda b, n: (b, 0, n)),
            pl.BlockSpec((1, C2, S), lambda b, n: (b, 0, 0)),
            pl.BlockSpec((H1, IN_CH), lambda b, n: (0, 0)),
            pl.BlockSpec((H1, 128), lambda b, n: (0, 0)),
        ],
        out_specs=[
            pl.BlockSpec((1, H1, NT), lambda b, n: (b, 0, n)),
            pl.BlockSpec((H1, 128), lambda b, n: (0, 0)),
            pl.BlockSpec((H1, 128), lambda b, n: (0, 0)),
        ],
        out_shape=[
            jax.ShapeDtypeStruct((B, H1, N), jnp.float32),
            jax.ShapeDtypeStruct((H1, 128), jnp.float32),
            jax.ShapeDtypeStruct((H1, 128), jnp.float32),
        ],
    )(x1p, x2tp, x1sq, x2sq, points1, points2, W1, _col(b1))

    mean1 = s1sum[:, 0] / cnt
    var1 = s1sq[:, 0] / cnt - mean1 * mean1
    scale1 = g1 / jnp.sqrt(var1 + eps)
    shift1 = be1 - mean1 * scale1

    h2, s2sum, s2sq = pl.pallas_call(
        _k2_body,
        grid=(B, NB),
        in_specs=[
            pl.BlockSpec((1, H1, NT), lambda b, n: (b, 0, n)),
            pl.BlockSpec((H1, 128), lambda b, n: (0, 0)),
            pl.BlockSpec((H1, 128), lambda b, n: (0, 0)),
            pl.BlockSpec((H2, H1), lambda b, n: (0, 0)),
            pl.BlockSpec((H2, 128), lambda b, n: (0, 0)),
        ],
        out_specs=[
            pl.BlockSpec((1, H2, NT), lambda b, n: (b, 0, n)),
            pl.BlockSpec((H2, 128), lambda b, n: (0, 0)),
            pl.BlockSpec((H2, 128), lambda b, n: (0, 0)),
        ],
        out_shape=[
            jax.ShapeDtypeStruct((B, H2, N), jnp.float32),
            jax.ShapeDtypeStruct((H2, 128), jnp.float32),
            jax.ShapeDtypeStruct((H2, 128), jnp.float32),
        ],
    )(h1, _col(scale1), _col(shift1), W2, _col(b2))

    mean2 = s2sum[:, 0] / cnt
    var2 = s2sq[:, 0] / cnt - mean2 * mean2
    scale2 = g2 / jnp.sqrt(var2 + eps)
    shift2 = be2 - mean2 * scale2

    out = pl.pallas_call(
        _k3_body,
        grid=(B, NB),
        in_specs=[
            pl.BlockSpec((1, H2, NT), lambda b, n: (b, 0, n)),
            pl.BlockSpec((H2, 128), lambda b, n: (0, 0)),
            pl.BlockSpec((H2, 128), lambda b, n: (0, 0)),
        ],
        out_specs=pl.BlockSpec((1, H2, NT), lambda b, n: (b, 0, n)),
        out_shape=jax.ShapeDtypeStruct((B, H2, N), jnp.float32),
    )(h2, _col(scale2), _col(shift2))

    return out
